# baseline (speedup 1.0000x reference)
import functools

import jax
import jax.numpy as jnp
from jax.experimental import pallas as pl

B_ = 16384
OBS = 128
HID = 64
NACT = 18
NQ = 32
BLK = 1024


def _body(x_ref, g0_ref, g1_ref, W1_ref, b1_ref, Wa_ref, ba_ref,
          Wopp_ref, bopp_ref, Wc1o_ref, wid_ref, wact_ref, bc1_ref,
          Wc2_ref, bc2_ref,
          val_ref, act_ref, alp_ref, oppp_ref, ents_ref, oents_ref):
    i = pl.program_id(0)
    blk = x_ref.shape[0]

    x = x_ref[...]
    W1 = W1_ref[...]
    b1 = b1_ref[...]
    f0 = jnp.tanh(jnp.dot(x[:, :OBS], W1, preferred_element_type=jnp.float32) + b1)
    f1 = jnp.tanh(jnp.dot(x[:, OBS:], W1, preferred_element_type=jnp.float32) + b1)

    Wa = Wa_ref[...]
    ba = ba_ref[...]
    Wopp = Wopp_ref[...]
    bopp = bopp_ref[...]
    iota = jax.lax.broadcasted_iota(jnp.int32, (blk, NACT), 1)

    acts = []
    lps = []
    ent_sum = jnp.zeros((), jnp.float32)
    oent_sum = jnp.zeros((), jnp.float32)
    for agent, (f, g_ref) in enumerate(((f0, g0_ref), (f1, g1_ref))):
        logits = jnp.dot(f, Wa, preferred_element_type=jnp.float32) + ba
        s = logits + g_ref[...]
        m = jnp.max(s, axis=-1, keepdims=True)
        a = jnp.min(jnp.where(s >= m, iota, NACT), axis=-1, keepdims=True)
        z = logits - jnp.max(logits, axis=-1, keepdims=True)
        logp = z - jnp.log(jnp.sum(jnp.exp(z), axis=-1, keepdims=True))
        lp_a = jnp.sum(jnp.where(iota == a, logp, 0.0), axis=-1, keepdims=True)
        ent_sum += -jnp.sum(jnp.exp(logp) * logp)
        ol = jnp.dot(f, Wopp, preferred_element_type=jnp.float32) + bopp
        oz = ol - jnp.max(ol, axis=-1, keepdims=True)
        ologp = oz - jnp.log(jnp.sum(jnp.exp(oz), axis=-1, keepdims=True))
        oent_sum += -jnp.sum(jnp.exp(ologp) * ologp)
        if agent == 1:
            oppp_ref[...] = jnp.exp(oz) / jnp.sum(jnp.exp(oz), axis=-1, keepdims=True)
        acts.append(a)
        lps.append(lp_a)

    a0, a1 = acts
    act_ref[...] = jnp.concatenate([a0, a1], axis=1)
    alp_ref[...] = jnp.concatenate(lps, axis=1)

    base = jnp.dot(x, Wc1o_ref[...], preferred_element_type=jnp.float32) + bc1_ref[...]
    wact = wact_ref[...]
    contrib = (a0.astype(jnp.float32) * wact[0:1, :]
               + a1.astype(jnp.float32) * wact[1:2, :])
    pre = base + contrib
    Wc2 = Wc2_ref[...]
    bc2 = bc2_ref[...]
    for agent, (h, a) in enumerate(((jnp.tanh(pre), a0),
                                    (jnp.tanh(pre + wid_ref[...]), a1))):
        v = jnp.dot(h, Wc2, preferred_element_type=jnp.float32) + bc2
        acc = jnp.zeros((blk, NQ), jnp.float32)
        for k in range(NACT):
            msk = (a == k).astype(jnp.float32)
            acc = acc + msk * v[:, k * NQ:(k + 1) * NQ]
        val_ref[agent, :, :] = acc

    @pl.when(i == 0)
    def _init():
        ents_ref[...] = jnp.zeros_like(ents_ref)
        oents_ref[...] = jnp.zeros_like(oents_ref)

    ents_ref[...] += jnp.reshape(ent_sum, (1, 1))
    oents_ref[...] += jnp.reshape(oent_sum, (1, 1))


@functools.partial(jax.jit, static_argnames=())
def kernel(inputs, rnn_hxs, masks, W1, b1, Wa, ba, Wopp, bopp, Wc1, bc1, Wc2, bc2):
    bsz = inputs.shape[0]
    x = inputs.reshape(bsz, 2 * OBS)
    skey = jax.random.key(42)
    g0 = jax.random.gumbel(jax.random.fold_in(skey, 0), (bsz, NACT), jnp.float32)
    g1 = jax.random.gumbel(jax.random.fold_in(skey, 1), (bsz, NACT), jnp.float32)

    Wc1o = Wc1[:2 * OBS]
    wid = Wc1[2 * OBS:2 * OBS + 1]
    wact = Wc1[2 * OBS + 1:]
    grid = (bsz // BLK,)

    def row_spec(width, dtype=None):
        return pl.BlockSpec((BLK, width), lambda i: (i, 0))

    def full_spec(shape):
        nd = len(shape)
        return pl.BlockSpec(shape, lambda i: (0,) * nd)

    out_shapes = (
        jax.ShapeDtypeStruct((2, bsz, NQ), jnp.float32),
        jax.ShapeDtypeStruct((bsz, 2), jnp.int32),
        jax.ShapeDtypeStruct((bsz, 2), jnp.float32),
        jax.ShapeDtypeStruct((bsz, NACT), jnp.float32),
        jax.ShapeDtypeStruct((1, 1), jnp.float32),
        jax.ShapeDtypeStruct((1, 1), jnp.float32),
    )
    out_specs = (
        pl.BlockSpec((2, BLK, NQ), lambda i: (0, i, 0)),
        row_spec(2),
        row_spec(2),
        row_spec(NACT),
        full_spec((1, 1)),
        full_spec((1, 1)),
    )
    in_specs = [
        row_spec(2 * OBS),
        row_spec(NACT),
        row_spec(NACT),
        full_spec((OBS, HID)),
        full_spec((1, HID)),
        full_spec((HID, NACT)),
        full_spec((1, NACT)),
        full_spec((HID, NACT)),
        full_spec((1, NACT)),
        full_spec((2 * OBS, HID)),
        full_spec((1, HID)),
        full_spec((2, HID)),
        full_spec((1, HID)),
        full_spec((HID, NACT * NQ)),
        full_spec((1, NACT * NQ)),
    ]

    value, action, alp, oppp, ents, oents = pl.pallas_call(
        _body,
        grid=grid,
        in_specs=in_specs,
        out_specs=out_specs,
        out_shape=out_shapes,
    )(x, g0, g1, W1, b1.reshape(1, HID), Wa, ba.reshape(1, NACT),
      Wopp, bopp.reshape(1, NACT), Wc1o, wid, wact, bc1.reshape(1, HID),
      Wc2, bc2.reshape(1, NACT * NQ))

    dist_entropy = ents[0, 0] * (0.5 / bsz)
    opp_dist_entropy = oents[0, 0] * (0.5 / bsz)
    opp_probs = oppp.reshape(bsz, 1, NACT)
    return (value, action, alp, dist_entropy, opp_probs, opp_dist_entropy, rnn_hxs)


# speedup vs baseline: 1.3582x; 1.3582x over previous
import functools

import jax
import jax.numpy as jnp
import numpy as np
from jax.experimental import pallas as pl

B_ = 16384
OBS = 128
HID = 64
NACT = 18
NQ = 32
BLK = 1024


def _body(x_ref, g0_ref, g1_ref, W1_ref, b1_ref, Wa_ref, ba_ref,
          Wopp_ref, bopp_ref, Wc1o_ref, wid_ref, wact_ref, bc1_ref,
          Wc2_ref, bc2_ref,
          val_ref, act_ref, alp_ref, oppp_ref, ents_ref, oents_ref):
    i = pl.program_id(0)
    blk = x_ref.shape[0]

    x = x_ref[...]
    W1 = W1_ref[...]
    b1 = b1_ref[...]
    f0 = jnp.tanh(jnp.dot(x[:, :OBS], W1, preferred_element_type=jnp.float32) + b1)
    f1 = jnp.tanh(jnp.dot(x[:, OBS:], W1, preferred_element_type=jnp.float32) + b1)

    Wa = Wa_ref[...]
    ba = ba_ref[...]
    Wopp = Wopp_ref[...]
    bopp = bopp_ref[...]
    iota = jax.lax.broadcasted_iota(jnp.int32, (blk, NACT), 1)

    acts = []
    lps = []
    ent_sum = jnp.zeros((), jnp.float32)
    oent_sum = jnp.zeros((), jnp.float32)
    for agent, (f, g_ref) in enumerate(((f0, g0_ref), (f1, g1_ref))):
        logits = jnp.dot(f, Wa, preferred_element_type=jnp.float32) + ba
        s = logits + g_ref[...]
        m = jnp.max(s, axis=-1, keepdims=True)
        a = jnp.min(jnp.where(s >= m, iota, NACT), axis=-1, keepdims=True)
        z = logits - jnp.max(logits, axis=-1, keepdims=True)
        logp = z - jnp.log(jnp.sum(jnp.exp(z), axis=-1, keepdims=True))
        lp_a = jnp.sum(jnp.where(iota == a, logp, 0.0), axis=-1, keepdims=True)
        ent_sum += -jnp.sum(jnp.exp(logp) * logp)
        ol = jnp.dot(f, Wopp, preferred_element_type=jnp.float32) + bopp
        oz = ol - jnp.max(ol, axis=-1, keepdims=True)
        ologp = oz - jnp.log(jnp.sum(jnp.exp(oz), axis=-1, keepdims=True))
        oent_sum += -jnp.sum(jnp.exp(ologp) * ologp)
        if agent == 1:
            oppp_ref[:, 0, :] = jnp.exp(oz) / jnp.sum(jnp.exp(oz), axis=-1, keepdims=True)
        acts.append(a)
        lps.append(lp_a)

    a0, a1 = acts
    act_ref[...] = jnp.concatenate([a0, a1], axis=1)
    alp_ref[...] = jnp.concatenate(lps, axis=1)

    base = jnp.dot(x, Wc1o_ref[...], preferred_element_type=jnp.float32) + bc1_ref[...]
    wact = wact_ref[...]
    contrib = (a0.astype(jnp.float32) * wact[0:1, :]
               + a1.astype(jnp.float32) * wact[1:2, :])
    pre = base + contrib
    Wc2 = Wc2_ref[...]
    bc2 = bc2_ref[...]
    for agent, (h, a) in enumerate(((jnp.tanh(pre), a0),
                                    (jnp.tanh(pre + wid_ref[...]), a1))):
        v = jnp.dot(h, Wc2, preferred_element_type=jnp.float32) + bc2
        acc = jnp.zeros((blk, NQ), jnp.float32)
        for k in range(NACT):
            msk = (a == k).astype(jnp.float32)
            acc = acc + msk * v[:, k * NQ:(k + 1) * NQ]
        val_ref[agent, :, :] = acc

    @pl.when(i == 0)
    def _init():
        ents_ref[...] = jnp.zeros_like(ents_ref)
        oents_ref[...] = jnp.zeros_like(oents_ref)

    ents_ref[...] += jnp.reshape(ent_sum, (1, 1))
    oents_ref[...] += jnp.reshape(oent_sum, (1, 1))


@functools.lru_cache(maxsize=4)
def _gumbel_const(bsz):
    with jax.ensure_compile_time_eval():
        skey = jax.random.key(42)
        g0 = jax.random.gumbel(jax.random.fold_in(skey, 0), (bsz, NACT), jnp.float32)
        g1 = jax.random.gumbel(jax.random.fold_in(skey, 1), (bsz, NACT), jnp.float32)
        return np.asarray(g0), np.asarray(g1)


@functools.partial(jax.jit, static_argnames=())
def kernel(inputs, rnn_hxs, masks, W1, b1, Wa, ba, Wopp, bopp, Wc1, bc1, Wc2, bc2):
    bsz = inputs.shape[0]
    x = inputs.reshape(bsz, 2 * OBS)
    g0, g1 = _gumbel_const(bsz)

    Wc1o = Wc1[:2 * OBS]
    wid = Wc1[2 * OBS:2 * OBS + 1]
    wact = Wc1[2 * OBS + 1:]
    grid = (bsz // BLK,)

    def row_spec(width, dtype=None):
        return pl.BlockSpec((BLK, width), lambda i: (i, 0))

    def full_spec(shape):
        nd = len(shape)
        return pl.BlockSpec(shape, lambda i: (0,) * nd)

    out_shapes = (
        jax.ShapeDtypeStruct((2, bsz, NQ), jnp.float32),
        jax.ShapeDtypeStruct((bsz, 2), jnp.int32),
        jax.ShapeDtypeStruct((bsz, 2), jnp.float32),
        jax.ShapeDtypeStruct((bsz, 1, NACT), jnp.float32),
        jax.ShapeDtypeStruct((1, 1), jnp.float32),
        jax.ShapeDtypeStruct((1, 1), jnp.float32),
    )
    out_specs = (
        pl.BlockSpec((2, BLK, NQ), lambda i: (0, i, 0)),
        row_spec(2),
        row_spec(2),
        pl.BlockSpec((BLK, 1, NACT), lambda i: (i, 0, 0)),
        full_spec((1, 1)),
        full_spec((1, 1)),
    )
    in_specs = [
        row_spec(2 * OBS),
        row_spec(NACT),
        row_spec(NACT),
        full_spec((OBS, HID)),
        full_spec((1, HID)),
        full_spec((HID, NACT)),
        full_spec((1, NACT)),
        full_spec((HID, NACT)),
        full_spec((1, NACT)),
        full_spec((2 * OBS, HID)),
        full_spec((1, HID)),
        full_spec((2, HID)),
        full_spec((1, HID)),
        full_spec((HID, NACT * NQ)),
        full_spec((1, NACT * NQ)),
    ]

    value, action, alp, oppp, ents, oents = pl.pallas_call(
        _body,
        grid=grid,
        in_specs=in_specs,
        out_specs=out_specs,
        out_shape=out_shapes,
    )(x, g0, g1, W1, b1.reshape(1, HID), Wa, ba.reshape(1, NACT),
      Wopp, bopp.reshape(1, NACT), Wc1o, wid, wact, bc1.reshape(1, HID),
      Wc2, bc2.reshape(1, NACT * NQ))

    dist_entropy = ents[0, 0] * (0.5 / bsz)
    opp_dist_entropy = oents[0, 0] * (0.5 / bsz)
    return (value, action, alp, dist_entropy, oppp, opp_dist_entropy, rnn_hxs)
